# baseline (speedup 1.0000x reference)
import functools
import jax
import jax.numpy as jnp
from jax import lax
from jax.experimental import pallas as pl
from jax.experimental.pallas import tpu as pltpu
from jax.experimental.pallas import tpu_sc as plsc

_NC = 2
_NS = 16
_NW = _NC * _NS
_LB = 128


def _leaky_relu(x):
    return jnp.where(x >= 0, x, 0.01 * x)


def _node_embed_body(nf_ref, wn_ref, bn_ref, hv_ref):
    hv_ref[...] = _leaky_relu(
        jnp.dot(nf_ref[...], wn_ref[...], preferred_element_type=jnp.float32)
        + bn_ref[...])


def _edge_body(nfsrc_ref, ef_ref, wet_ref, web_ref, be_ref, wn_ref, bn_ref,
               wmsg_ref, bmsg_ref, w1_ref, w2_ref, bl_ref, em_ref, ex_ref):
    nfs = nfsrc_ref[...]
    q = jnp.dot(ef_ref[...], web_ref[...], preferred_element_type=jnp.float32) + be_ref[...]
    h_wv = _leaky_relu(jnp.dot(nfs, wet_ref[...], preferred_element_type=jnp.float32) + q)
    m = jnp.dot(h_wv, wmsg_ref[...], preferred_element_type=jnp.float32) + bmsg_ref[...]
    h_vsrc = _leaky_relu(jnp.dot(nfs, wn_ref[...], preferred_element_type=jnp.float32)
                         + bn_ref[...])
    t = (jnp.dot(h_vsrc, w1_ref[...], preferred_element_type=jnp.float32)
         + jnp.dot(h_wv, w2_ref[...], preferred_element_type=jnp.float32))
    logit = _leaky_relu(t + bl_ref[...])
    ex = jnp.exp(logit)
    ex_ref[...] = jnp.concatenate([ex, jnp.zeros((ex.shape[0], 15), jnp.float32)], axis=1)
    em_ref[...] = ex * m


def _gru_body(s0_ref, s1_ref, den_ref, hv_ref, wih_t_ref, bih_ref, whh_t_ref,
              bhh_ref, out_ref):
    den = den_ref[...]
    den = jnp.where(den > 0, den, 1.0)
    c = (s0_ref[...] + s1_ref[...]) / den
    c = jnp.where(c >= 0, c, jnp.exp(jnp.minimum(c, 0.0)) - 1.0)
    gi = jnp.dot(c, wih_t_ref[...], preferred_element_type=jnp.float32) + bih_ref[...]
    gh = jnp.dot(hv_ref[...], whh_t_ref[...], preferred_element_type=jnp.float32) + bhh_ref[...]
    nh = out_ref.shape[1]
    i_r = gi[:, :nh]; i_z = gi[:, nh:2 * nh]; i_n = gi[:, 2 * nh:]
    h_r = gh[:, :nh]; h_z = gh[:, nh:2 * nh]; h_n = gh[:, 2 * nh:]
    r = jax.nn.sigmoid(i_r + h_r)
    z = jax.nn.sigmoid(i_z + h_z)
    n = jnp.tanh(i_n + r * h_n)
    hv = hv_ref[...]
    h_new = (1.0 - z) * n + z * hv
    out_ref[...] = jnp.maximum(h_new, 0.0)


def _sc_gather_body(KB, nf_hbm, src2d_hbm, nfsrc_hbm, idx_v, rows_v, sem):
    cid = lax.axis_index("c")
    sid = lax.axis_index("s")
    wid = sid * _NC + cid
    rbase = wid * KB
    pltpu.sync_copy(src2d_hbm.at[pl.ds(rbase, KB)], idx_v)

    def body(j, carry):
        pltpu.async_copy(nf_hbm.at[idx_v.at[j]], rows_v, sem).wait()
        pltpu.sync_copy(rows_v, nfsrc_hbm.at[pl.ds((rbase + j) * _LB, _LB)])
        return carry

    lax.fori_loop(0, KB, body, 0)


def _sc_scatter_body(KB, NSH, z_hbm, em_hbm, dst2d_hbm, sout_hbm,
                     idx_v, em_buf, sharedS, sem):
    cid = lax.axis_index("c")
    sid = lax.axis_index("s")
    wid = sid * _NC + cid
    rows_per_tile = NSH // _NS
    tbase = sid * rows_per_tile

    pltpu.sync_copy(z_hbm, em_buf)

    def zcp(i, c):
        pltpu.sync_copy(em_buf.at[pl.ds(0, 8)], sharedS.at[pl.ds(tbase + i * 8, 8)])
        return c
    lax.fori_loop(0, rows_per_tile // 8, zcp, 0)
    plsc.subcore_barrier()

    pltpu.sync_copy(dst2d_hbm.at[pl.ds(wid * KB, KB)], idx_v)

    def body(j, c2):
        pltpu.sync_copy(em_hbm.at[pl.ds((wid * KB + j) * _LB, _LB)], em_buf)
        pltpu.sync_copy(em_buf, sharedS.at[idx_v.at[j]], add=True)
        return c2
    lax.fori_loop(0, KB, body, 0)
    plsc.subcore_barrier()

    def rb(i, c):
        pltpu.sync_copy(sharedS.at[pl.ds(tbase + i * 8, 8)], em_buf.at[pl.ds(0, 8)])
        pltpu.sync_copy(em_buf.at[pl.ds(0, 8)],
                        sout_hbm.at[pl.ds(cid * NSH + tbase + i * 8, 8)])
        return c
    lax.fori_loop(0, rows_per_tile // 8, rb, 0)


def kernel(node_feats, edge_feats, edge_index, W_node, b_node, W_edge, b_edge,
           W_logit, b_logit, W_msg, b_msg, W_ih, b_ih, W_hh, b_hh):
    N, DF = node_feats.shape
    E, DE = edge_feats.shape
    NH = W_node.shape[1]
    CS = W_msg.shape[1]
    src = edge_index[0].astype(jnp.int32)
    dst = edge_index[1].astype(jnp.int32)

    BN = 1000
    BE = 4000

    w1 = W_logit[:NH]
    w2 = W_logit[NH:]
    wet = W_edge[:DF]
    web = W_edge[DF:]

    hv = pl.pallas_call(
        _node_embed_body,
        grid=(N // BN,),
        in_specs=[
            pl.BlockSpec((BN, DF), lambda i: (i, 0)),
            pl.BlockSpec((DF, NH), lambda i: (0, 0)),
            pl.BlockSpec((1, NH), lambda i: (0, 0)),
        ],
        out_specs=pl.BlockSpec((BN, NH), lambda i: (i, 0)),
        out_shape=jax.ShapeDtypeStruct((N, NH), jnp.float32),
            )(node_feats, W_node, b_node.reshape(1, NH))

    KB = ((-(-E // (_NW * _LB))) + 7) // 8 * 8
    E2 = _NW * KB * _LB
    NSH = 10112
    pad = E2 - E
    src2d = jnp.concatenate([src, jnp.zeros((pad,), jnp.int32)]).reshape(E2 // _LB, _LB)
    dst2d = jnp.concatenate([dst, jnp.full((pad,), N, jnp.int32)]).reshape(E2 // _LB, _LB)

    mesh = plsc.VectorSubcoreMesh(core_axis_name="c", subcore_axis_name="s")
    NFsrc = pl.kernel(
            functools.partial(_sc_gather_body, KB),
            out_type=jax.ShapeDtypeStruct((E2, DF), jnp.float32),
            mesh=mesh,
            scratch_types=[
                pltpu.VMEM((KB, _LB), jnp.int32),
                pltpu.VMEM((_LB, DF), jnp.float32),
                pltpu.SemaphoreType.DMA,
            ],
    )(node_feats, src2d)

    em, ex = pl.pallas_call(
        _edge_body,
        grid=(E // BE,),
        in_specs=[
            pl.BlockSpec((BE, DF), lambda i: (i, 0)),
            pl.BlockSpec((BE, DE), lambda i: (i, 0)),
            pl.BlockSpec((DF, NH), lambda i: (0, 0)),
            pl.BlockSpec((DE, NH), lambda i: (0, 0)),
            pl.BlockSpec((1, NH), lambda i: (0, 0)),
            pl.BlockSpec((DF, NH), lambda i: (0, 0)),
            pl.BlockSpec((1, NH), lambda i: (0, 0)),
            pl.BlockSpec((NH, CS), lambda i: (0, 0)),
            pl.BlockSpec((1, CS), lambda i: (0, 0)),
            pl.BlockSpec((NH, 1), lambda i: (0, 0)),
            pl.BlockSpec((NH, 1), lambda i: (0, 0)),
            pl.BlockSpec((1, 1), lambda i: (0, 0)),
        ],
        out_specs=[
            pl.BlockSpec((BE, CS), lambda i: (i, 0)),
            pl.BlockSpec((BE, 16), lambda i: (i, 0)),
        ],
        out_shape=[
            jax.ShapeDtypeStruct((E2, CS), jnp.float32),
            jax.ShapeDtypeStruct((E2, 16), jnp.float32),
        ],
    )(NFsrc, edge_feats, wet, web, b_edge.reshape(1, NH),
      W_node, b_node.reshape(1, NH), W_msg, b_msg.reshape(1, CS),
      w1, w2, b_logit.reshape(1, 1))

    zrows = jnp.zeros((_LB, CS), jnp.float32)
    Sflat = pl.kernel(
            functools.partial(_sc_scatter_body, KB, NSH),
            out_type=jax.ShapeDtypeStruct((2 * NSH, CS), jnp.float32),
            mesh=mesh,
            scratch_types=[
                pltpu.VMEM((KB, _LB), jnp.int32),
                pltpu.VMEM((_LB, CS), jnp.float32),
                pltpu.VMEM_SHARED((NSH, CS), jnp.float32),
                pltpu.SemaphoreType.DMA,
            ],
    )(zrows, em, dst2d)
    S0 = Sflat[:N]
    S1 = Sflat[NSH:NSH + N]
    den = jax.ops.segment_sum(ex[:E, 0], dst, num_segments=N).reshape(N, 1)

    out = pl.pallas_call(
        _gru_body,
        grid=(N // BN,),
        in_specs=[
            pl.BlockSpec((BN, CS), lambda i: (i, 0)),
            pl.BlockSpec((BN, CS), lambda i: (i, 0)),
            pl.BlockSpec((BN, 1), lambda i: (i, 0)),
            pl.BlockSpec((BN, NH), lambda i: (i, 0)),
            pl.BlockSpec((CS, 3 * NH), lambda i: (0, 0)),
            pl.BlockSpec((1, 3 * NH), lambda i: (0, 0)),
            pl.BlockSpec((NH, 3 * NH), lambda i: (0, 0)),
            pl.BlockSpec((1, 3 * NH), lambda i: (0, 0)),
        ],
        out_specs=pl.BlockSpec((BN, NH), lambda i: (i, 0)),
        out_shape=jax.ShapeDtypeStruct((N, NH), jnp.float32),
            )(S0, S1, den, hv, W_ih.T, b_ih.reshape(1, 3 * NH), W_hh.T, b_hh.reshape(1, 3 * NH))

    return (out, edge_feats)


# speedup vs baseline: 1.0538x; 1.0538x over previous
import functools
import jax
import jax.numpy as jnp
from jax import lax
from jax.experimental import pallas as pl
from jax.experimental.pallas import tpu as pltpu
from jax.experimental.pallas import tpu_sc as plsc

_NC = 2
_NS = 16
_NW = _NC * _NS
_LB = 128


def _leaky_relu(x):
    return jnp.where(x >= 0, x, 0.01 * x)


def _node_embed_body(nf_ref, wn_ref, bn_ref, hv_ref):
    hv_ref[...] = _leaky_relu(
        jnp.dot(nf_ref[...], wn_ref[...], preferred_element_type=jnp.float32)
        + bn_ref[...])


def _edge_body(nfsrc_ref, ef_ref, wet_ref, web_ref, be_ref, wn_ref, bn_ref,
               wmsg_ref, bmsg_ref, w1_ref, w2_ref, bl_ref, em_ref, ex_ref):
    nfs = nfsrc_ref[...]
    q = jnp.dot(ef_ref[...], web_ref[...], preferred_element_type=jnp.float32) + be_ref[...]
    h_wv = _leaky_relu(jnp.dot(nfs, wet_ref[...], preferred_element_type=jnp.float32) + q)
    m = jnp.dot(h_wv, wmsg_ref[...], preferred_element_type=jnp.float32) + bmsg_ref[...]
    h_vsrc = _leaky_relu(jnp.dot(nfs, wn_ref[...], preferred_element_type=jnp.float32)
                         + bn_ref[...])
    t = (jnp.dot(h_vsrc, w1_ref[...], preferred_element_type=jnp.float32)
         + jnp.dot(h_wv, w2_ref[...], preferred_element_type=jnp.float32))
    logit = _leaky_relu(t + bl_ref[...])
    ex = jnp.exp(logit)
    ex_ref[...] = jnp.concatenate([ex, jnp.zeros((ex.shape[0], 15), jnp.float32)], axis=1)
    em_ref[...] = ex * m


def _gru_body(s0_ref, s1_ref, den_ref, hv_ref, wih_t_ref, bih_ref, whh_t_ref,
              bhh_ref, out_ref):
    den = den_ref[...]
    den = jnp.where(den > 0, den, 1.0)
    c = (s0_ref[...] + s1_ref[...]) / den
    c = jnp.where(c >= 0, c, jnp.exp(jnp.minimum(c, 0.0)) - 1.0)
    gi = jnp.dot(c, wih_t_ref[...], preferred_element_type=jnp.float32) + bih_ref[...]
    gh = jnp.dot(hv_ref[...], whh_t_ref[...], preferred_element_type=jnp.float32) + bhh_ref[...]
    nh = out_ref.shape[1]
    i_r = gi[:, :nh]; i_z = gi[:, nh:2 * nh]; i_n = gi[:, 2 * nh:]
    h_r = gh[:, :nh]; h_z = gh[:, nh:2 * nh]; h_n = gh[:, 2 * nh:]
    r = jax.nn.sigmoid(i_r + h_r)
    z = jax.nn.sigmoid(i_z + h_z)
    n = jnp.tanh(i_n + r * h_n)
    hv = hv_ref[...]
    h_new = (1.0 - z) * n + z * hv
    out_ref[...] = jnp.maximum(h_new, 0.0)


def _sc_gather_body(KB, nf_hbm, src2d_hbm, nfsrc_hbm, idx_v, rows_a, rows_b,
                    gsem_a, gsem_b, wsem_a, wsem_b):
    cid = lax.axis_index("c")
    sid = lax.axis_index("s")
    wid = sid * _NC + cid
    rbase = wid * KB
    pltpu.sync_copy(src2d_hbm.at[pl.ds(rbase, KB)], idx_v)
    bufs = (rows_a, rows_b)
    gsems = (gsem_a, gsem_b)
    wsems = (wsem_a, wsem_b)

    def pair(i, carry):
        for b in range(2):
            j = 2 * i + b
            buf, gs, ws = bufs[b], gsems[b], wsems[b]

            @pl.when(i > 0)
            def _():
                pltpu.make_async_copy(buf, nfsrc_hbm.at[pl.ds(0, _LB)], ws).wait()

            pltpu.async_copy(nf_hbm.at[idx_v.at[j]], buf, gs).wait()
            pltpu.async_copy(buf, nfsrc_hbm.at[pl.ds((rbase + j) * _LB, _LB)], ws)
        return carry

    lax.fori_loop(0, KB // 2, pair, 0)
    pltpu.make_async_copy(rows_a, nfsrc_hbm.at[pl.ds(0, _LB)], wsem_a).wait()
    pltpu.make_async_copy(rows_b, nfsrc_hbm.at[pl.ds(0, _LB)], wsem_b).wait()


def _sc_scatter_body(KB, NSH, z_hbm, em_hbm, dst2d_hbm, sout_hbm,
                     idx_v, em_buf, em_buf2, sharedS, sem, lsem, lsem2):
    cid = lax.axis_index("c")
    sid = lax.axis_index("s")
    wid = sid * _NC + cid
    rows_per_tile = NSH // _NS
    tbase = sid * rows_per_tile

    pltpu.sync_copy(z_hbm, em_buf)

    def zcp(i, c):
        pltpu.sync_copy(em_buf.at[pl.ds(0, 8)], sharedS.at[pl.ds(tbase + i * 8, 8)])
        return c
    lax.fori_loop(0, rows_per_tile // 8, zcp, 0)
    plsc.subcore_barrier()

    pltpu.sync_copy(dst2d_hbm.at[pl.ds(wid * KB, KB)], idx_v)

    pltpu.sync_copy(em_hbm.at[pl.ds(wid * KB * _LB, _LB)], em_buf)

    def pair(i, c2):
        j = 2 * i
        cpb = pltpu.async_copy(em_hbm.at[pl.ds((wid * KB + j + 1) * _LB, _LB)],
                               em_buf2, lsem)
        pltpu.sync_copy(em_buf, sharedS.at[idx_v.at[j]], add=True)
        cpb.wait()

        @pl.when(j + 2 < KB)
        def _():
            pltpu.async_copy(em_hbm.at[pl.ds((wid * KB + j + 2) * _LB, _LB)],
                             em_buf, lsem2)
        pltpu.sync_copy(em_buf2, sharedS.at[idx_v.at[j + 1]], add=True)

        @pl.when(j + 2 < KB)
        def _():
            pltpu.make_async_copy(em_hbm.at[pl.ds(0, _LB)], em_buf, lsem2).wait()
        return c2
    lax.fori_loop(0, KB // 2, pair, 0)
    plsc.subcore_barrier()

    def rb(i, c):
        pltpu.sync_copy(sharedS.at[pl.ds(tbase + i * 8, 8)], em_buf.at[pl.ds(0, 8)])
        pltpu.sync_copy(em_buf.at[pl.ds(0, 8)],
                        sout_hbm.at[pl.ds(cid * NSH + tbase + i * 8, 8)])
        return c
    lax.fori_loop(0, rows_per_tile // 8, rb, 0)


def kernel(node_feats, edge_feats, edge_index, W_node, b_node, W_edge, b_edge,
           W_logit, b_logit, W_msg, b_msg, W_ih, b_ih, W_hh, b_hh):
    N, DF = node_feats.shape
    E, DE = edge_feats.shape
    NH = W_node.shape[1]
    CS = W_msg.shape[1]
    src = edge_index[0].astype(jnp.int32)
    dst = edge_index[1].astype(jnp.int32)

    BN = 1000
    BE = 4000

    w1 = W_logit[:NH]
    w2 = W_logit[NH:]
    wet = W_edge[:DF]
    web = W_edge[DF:]

    hv = pl.pallas_call(
        _node_embed_body,
        grid=(N // BN,),
        in_specs=[
            pl.BlockSpec((BN, DF), lambda i: (i, 0)),
            pl.BlockSpec((DF, NH), lambda i: (0, 0)),
            pl.BlockSpec((1, NH), lambda i: (0, 0)),
        ],
        out_specs=pl.BlockSpec((BN, NH), lambda i: (i, 0)),
        out_shape=jax.ShapeDtypeStruct((N, NH), jnp.float32),
            )(node_feats, W_node, b_node.reshape(1, NH))

    KB = ((-(-E // (_NW * _LB))) + 7) // 8 * 8
    E2 = _NW * KB * _LB
    NSH = 10112
    pad = E2 - E
    src2d = jnp.concatenate([src, jnp.zeros((pad,), jnp.int32)]).reshape(E2 // _LB, _LB)
    dst2d = jnp.concatenate([dst, jnp.full((pad,), N, jnp.int32)]).reshape(E2 // _LB, _LB)

    mesh = plsc.VectorSubcoreMesh(core_axis_name="c", subcore_axis_name="s")
    NFsrc = pl.kernel(
            functools.partial(_sc_gather_body, KB),
            out_type=jax.ShapeDtypeStruct((E2, DF), jnp.float32),
            mesh=mesh,
            scratch_types=[
                pltpu.VMEM((KB, _LB), jnp.int32),
                pltpu.VMEM((_LB, DF), jnp.float32),
                pltpu.VMEM((_LB, DF), jnp.float32),
                pltpu.SemaphoreType.DMA,
                pltpu.SemaphoreType.DMA,
                pltpu.SemaphoreType.DMA,
                pltpu.SemaphoreType.DMA,
            ],
    )(node_feats, src2d)

    em, ex = pl.pallas_call(
        _edge_body,
        grid=(E // BE,),
        in_specs=[
            pl.BlockSpec((BE, DF), lambda i: (i, 0)),
            pl.BlockSpec((BE, DE), lambda i: (i, 0)),
            pl.BlockSpec((DF, NH), lambda i: (0, 0)),
            pl.BlockSpec((DE, NH), lambda i: (0, 0)),
            pl.BlockSpec((1, NH), lambda i: (0, 0)),
            pl.BlockSpec((DF, NH), lambda i: (0, 0)),
            pl.BlockSpec((1, NH), lambda i: (0, 0)),
            pl.BlockSpec((NH, CS), lambda i: (0, 0)),
            pl.BlockSpec((1, CS), lambda i: (0, 0)),
            pl.BlockSpec((NH, 1), lambda i: (0, 0)),
            pl.BlockSpec((NH, 1), lambda i: (0, 0)),
            pl.BlockSpec((1, 1), lambda i: (0, 0)),
        ],
        out_specs=[
            pl.BlockSpec((BE, CS), lambda i: (i, 0)),
            pl.BlockSpec((BE, 16), lambda i: (i, 0)),
        ],
        out_shape=[
            jax.ShapeDtypeStruct((E2, CS), jnp.float32),
            jax.ShapeDtypeStruct((E2, 16), jnp.float32),
        ],
    )(NFsrc, edge_feats, wet, web, b_edge.reshape(1, NH),
      W_node, b_node.reshape(1, NH), W_msg, b_msg.reshape(1, CS),
      w1, w2, b_logit.reshape(1, 1))

    zrows = jnp.zeros((_LB, CS), jnp.float32)
    Sflat = pl.kernel(
            functools.partial(_sc_scatter_body, KB, NSH),
            out_type=jax.ShapeDtypeStruct((2 * NSH, CS), jnp.float32),
            mesh=mesh,
            scratch_types=[
                pltpu.VMEM((KB, _LB), jnp.int32),
                pltpu.VMEM((_LB, CS), jnp.float32),
                pltpu.VMEM((_LB, CS), jnp.float32),
                pltpu.VMEM_SHARED((NSH, CS), jnp.float32),
                pltpu.SemaphoreType.DMA,
                pltpu.SemaphoreType.DMA,
                pltpu.SemaphoreType.DMA,
            ],
    )(zrows, em, dst2d)
    S0 = Sflat[:N]
    S1 = Sflat[NSH:NSH + N]
    den = jax.ops.segment_sum(ex[:E, 0], dst, num_segments=N).reshape(N, 1)

    out = pl.pallas_call(
        _gru_body,
        grid=(N // BN,),
        in_specs=[
            pl.BlockSpec((BN, CS), lambda i: (i, 0)),
            pl.BlockSpec((BN, CS), lambda i: (i, 0)),
            pl.BlockSpec((BN, 1), lambda i: (i, 0)),
            pl.BlockSpec((BN, NH), lambda i: (i, 0)),
            pl.BlockSpec((CS, 3 * NH), lambda i: (0, 0)),
            pl.BlockSpec((1, 3 * NH), lambda i: (0, 0)),
            pl.BlockSpec((NH, 3 * NH), lambda i: (0, 0)),
            pl.BlockSpec((1, 3 * NH), lambda i: (0, 0)),
        ],
        out_specs=pl.BlockSpec((BN, NH), lambda i: (i, 0)),
        out_shape=jax.ShapeDtypeStruct((N, NH), jnp.float32),
            )(S0, S1, den, hv, W_ih.T, b_ih.reshape(1, 3 * NH), W_hh.T, b_hh.reshape(1, 3 * NH))

    return (out, edge_feats)
